# Initial kernel scaffold; baseline (speedup 1.0000x reference)
import jax
import jax.numpy as jnp
from jax.experimental import pallas as pl


def kernel(x, edge_index, W1, b1, W2, b2):
    raise NotImplementedError("write your pallas kernel here")


# speedup vs baseline: 6.3605x; 6.3605x over previous
import functools

import jax
import jax.numpy as jnp
from jax import lax
from jax.experimental import pallas as pl
from jax.experimental.pallas import tpu as pltpu
from jax.experimental.pallas import tpu_sc as plsc

N = 50000
E = 800000
C = 64
HC = 32
H = 256
NC, NS = 2, 16
BLK = 128
UNROLL = 4
CHUNK = BLK * UNROLL
EPT = 50176
NCHUNKS = EPT // CHUNK
EPAD = EPT * NS
IDXROWS = EPAD // BLK
ROWS_PT = IDXROWS // NS
ACC_ROWS = 51200
APT = ACC_ROWS // NS
NOUT = 50048
WPT = NOUT // NS
XROWS = 50008


def _sc_segsum(x_lo, x_hi, src2d, dst2d):
    mesh = plsc.VectorSubcoreMesh(core_axis_name="c", subcore_axis_name="s")

    @functools.partial(
        pl.kernel,
        out_type=jax.ShapeDtypeStruct((2, 2 * NOUT, HC), jnp.float32),
        mesh=mesh,
        scratch_types=[
            pltpu.VMEM_SHARED((ACC_ROWS, HC), jnp.float32),
            pltpu.VMEM((CHUNK, HC), jnp.float32),
            pltpu.VMEM((UNROLL, BLK), jnp.int32),
            pltpu.VMEM((UNROLL, BLK), jnp.int32),
            pltpu.SemaphoreType.DMA,
            pltpu.SemaphoreType.DMA,
        ],
        compiler_params=pltpu.CompilerParams(use_tc_tiling_on_sc=False),
    )
    def seg_kernel(xlo_hbm, xhi_hbm, src_hbm, dst_hbm, out_hbm,
                   acc, rows, gidx, sidx, gsem, ssem):
        c = lax.axis_index("c")
        s = lax.axis_index("s")

        for d in range(2):
            g_hbm = src_hbm if d == 0 else dst_hbm
            s_hbm = dst_hbm if d == 0 else src_hbm

            def zrow(i, z):
                rows[i, pl.ds(0, 16)] = jnp.zeros((16,), jnp.float32)
                rows[i, pl.ds(16, 16)] = jnp.zeros((16,), jnp.float32)
                return z
            lax.fori_loop(0, CHUNK, zrow, 0)
            zbase = s * APT
            zoff = 0
            while zoff < APT:
                zn = min(CHUNK, APT - zoff)
                pltpu.sync_copy(rows.at[pl.ds(0, zn)],
                                acc.at[pl.ds(zbase + zoff, zn)])
                zoff += zn
            plsc.subcore_barrier()

            def chunk(kk, carry):
                row0 = s * ROWS_PT + kk * UNROLL
                pltpu.sync_copy(g_hbm.at[pl.ds(row0, UNROLL)], gidx)
                pltpu.sync_copy(s_hbm.at[pl.ds(row0, UNROLL)], sidx)

                @pl.when(c == 0)
                def _():
                    descs = [
                        pltpu.async_copy(xlo_hbm.at[gidx.at[j]],
                                         rows.at[pl.ds(j * BLK, BLK)], gsem)
                        for j in range(UNROLL)
                    ]
                    for dd in descs:
                        dd.wait()

                @pl.when(c == 1)
                def _():
                    descs = [
                        pltpu.async_copy(xhi_hbm.at[gidx.at[j]],
                                         rows.at[pl.ds(j * BLK, BLK)], gsem)
                        for j in range(UNROLL)
                    ]
                    for dd in descs:
                        dd.wait()

                sdescs = [
                    pltpu.async_copy(rows.at[pl.ds(j * BLK, BLK)],
                                     acc.at[sidx.at[j]], ssem, add=True)
                    for j in range(UNROLL)
                ]
                for dd in sdescs:
                    dd.wait()
                return carry
            lax.fori_loop(0, NCHUNKS, chunk, 0)
            plsc.subcore_barrier()

            pltpu.sync_copy(
                acc.at[pl.ds(s * WPT, WPT)],
                out_hbm.at[d].at[pl.ds(c * NOUT + s * WPT, WPT)],
            )
            plsc.subcore_barrier()

    return seg_kernel(x_lo, x_hi, src2d, dst2d)


def _mlp(out4, W1r, b1, W2, b2):
    B = 2000

    def body(a_ref, w1_ref, b1_ref, w2_ref, b2_ref, o_ref):
        h1 = (
            jnp.dot(a_ref[0, 0], w1_ref[0, 0], preferred_element_type=jnp.float32)
            + jnp.dot(a_ref[0, 1], w1_ref[0, 1], preferred_element_type=jnp.float32)
            + jnp.dot(a_ref[1, 0], w1_ref[1, 0], preferred_element_type=jnp.float32)
            + jnp.dot(a_ref[1, 1], w1_ref[1, 1], preferred_element_type=jnp.float32)
            + b1_ref[...]
        )
        h1 = jnp.maximum(h1, 0.0)
        o_ref[...] = (
            jnp.dot(h1, w2_ref[...], preferred_element_type=jnp.float32)
            + b2_ref[...]
        )

    return pl.pallas_call(
        body,
        grid=(N // B,),
        in_specs=[
            pl.BlockSpec((2, 2, B, HC), lambda i: (0, 0, i, 0)),
            pl.BlockSpec((2, 2, HC, H), lambda i: (0, 0, 0, 0)),
            pl.BlockSpec((1, H), lambda i: (0, 0)),
            pl.BlockSpec((H, C), lambda i: (0, 0)),
            pl.BlockSpec((1, C), lambda i: (0, 0)),
        ],
        out_specs=pl.BlockSpec((B, C), lambda i: (i, 0)),
        out_shape=jax.ShapeDtypeStruct((N, C), jnp.float32),
    )(out4, W1r, b1.reshape(1, H), W2, b2.reshape(1, C))


def kernel(x, edge_index, W1, b1, W2, b2):
    src = edge_index[0].astype(jnp.int32)
    dst = edge_index[1].astype(jnp.int32)
    pad = jnp.full((EPAD - E,), N, jnp.int32)
    src2d = jnp.concatenate([src, pad]).reshape(IDXROWS, BLK)
    dst2d = jnp.concatenate([dst, pad]).reshape(IDXROWS, BLK)
    x_lo = jnp.pad(x[:, :HC], ((0, XROWS - N), (0, 0)))
    x_hi = jnp.pad(x[:, HC:], ((0, XROWS - N), (0, 0)))
    out = _sc_segsum(x_lo, x_hi, src2d, dst2d)
    out4 = out.reshape(2, 2, NOUT, HC)[:, :, :N, :]
    return _mlp(out4, W1.reshape(2, 2, HC, H), b1, W2, b2)
